# baseline (speedup 1.0000x reference)
import functools
import jax
import jax.numpy as jnp
from jax import lax
from jax.experimental import pallas as pl
from jax.experimental.pallas import tpu as pltpu
from jax.experimental.pallas import tpu_sc as plsc

N = 100000
K = 50000
D = 128
NC, NS = 2, 16
W = NC * NS
CH = 128
NCH = 13
R = CH * NCH
RING = 5

_mesh = plsc.VectorSubcoreMesh(core_axis_name="c", subcore_axis_name="s")


def _wid():
    return lax.axis_index("s") * NC + lax.axis_index("c")


def _win_start(w):
    return jnp.minimum(w * R, K - R).astype(jnp.int32)


def _vmax(vec):
    r = vec[0]
    for i in range(1, 16):
        r = jnp.maximum(r, vec[i])
    return r


def _vsum(vec):
    r = vec[0]
    for i in range(1, 16):
        r = r + vec[i]
    return r


def _gather_body(idx_hbm, qf_hbm, qp_hbm, qf_out, qp_out,
                 idx1d, rows, qpidx, qpg, qpi, semG, semW, semP):
    w = _wid()
    o = _win_start(w)
    iota = lax.iota(jnp.int32, 16)

    pltpu.sync_copy(idx_hbm.at[pl.ds(o, R)], idx1d)

    def pstep(i, carry):
        v = idx1d[pl.ds(i * 16, 16)] * 3
        qpidx[pl.ds(i * 16, 16)] = v
        qpidx[pl.ds(R + i * 16, 16)] = v + 1
        qpidx[pl.ds(2 * R + i * 16, 16)] = v + 2
        return carry

    lax.fori_loop(0, R // 16, pstep, jnp.int32(0))
    gp = [pltpu.async_copy(
              qp_hbm.at[qpidx.at[pl.ds(comp * R + c * CH, CH)]],
              qpg.at[pl.ds(comp * R + c * CH, CH)], semP)
          for comp in range(3) for c in range(NCH)]

    gq, sq = [None] * NCH, [None] * NCH
    for c in range(4):
        gq[c] = pltpu.async_copy(qf_hbm.at[idx1d.at[pl.ds(c * CH, CH)]],
                                 rows.at[c % RING], semG)
    for c in range(NCH):
        gq[c].wait()
        sq[c] = pltpu.async_copy(rows.at[c % RING],
                                 qf_out.at[pl.ds(o + c * CH, CH)], semW)
        nxt = c + 4
        if nxt < NCH:
            prv = nxt - RING
            if prv >= 0:
                sq[prv].wait()
            gq[nxt] = pltpu.async_copy(
                qf_hbm.at[idx1d.at[pl.ds(nxt * CH, CH)]],
                rows.at[nxt % RING], semG)
    for c in range(max(0, NCH - RING), NCH):
        sq[c].wait()

    for g in gp:
        g.wait()

    def istep(i, carry):
        base = i * 16
        pos = (iota + base) * 3
        plsc.store_scatter(qpi, [pos], qpg[pl.ds(base, 16)])
        plsc.store_scatter(qpi, [pos + 1], qpg[pl.ds(R + base, 16)])
        plsc.store_scatter(qpi, [pos + 2], qpg[pl.ds(2 * R + base, 16)])
        return carry

    lax.fori_loop(0, R // 16, istep, jnp.int32(0))
    pltpu.sync_copy(qpi, qp_out.at[pl.ds(3 * o, 3 * R)])


_sc_params = pltpu.CompilerParams(needs_layout_passes=False)

NP = 102400
NW = 16
C2 = NP // NW
SEG2 = C2 // 16
BINS = 256
SCCH = 128
NSL = 8

_I32MIN = -2147483648

_mesh1 = plsc.VectorSubcoreMesh(core_axis_name="c", subcore_axis_name="s",
                                num_cores=1)


def _digit(kv, shift):
    d = kv
    if shift:
        d = plsc.bitcast(
            plsc.bitcast(d, jnp.uint32) >> jnp.uint32(shift), jnp.int32)
    return d & jnp.int32(BINS - 1)


def _sort_body(attn_hbm, idx_out, qa_out,
               abuf, tbuf, kbuf, vbuf, hist, cnt, redb, totb, wpreb, baseb,
               credv, posbuf, qatail, s0, s1, s2, s3, s4, s5, s6, s7,
               skA, siA, skB, siB, cshared, semK, semV):
    slots = [s0, s1, s2, s3, s4, s5, s6, s7]
    w = lax.axis_index("s")
    base = w * C2
    iota = lax.iota(jnp.int32, 16)
    ones = jnp.ones((16,), jnp.int32)
    lanebase = iota * BINS

    last = NW - 1
    full_len = N - last * C2
    al = (full_len // SCCH) * SCCH

    @pl.when(w < last)
    def _():
        pltpu.sync_copy(attn_hbm.at[pl.ds(base, C2)], abuf)

    @pl.when(w == last)
    def _():
        pltpu.sync_copy(attn_hbm.at[pl.ds(last * C2, al)],
                        abuf.at[pl.ds(0, al)])
        pltpu.sync_copy(attn_hbm.at[pl.ds(N - SCCH, SCCH)], tbuf)
        ninf = jnp.full((16,), -jnp.inf, jnp.float32)
        for v in range(SCCH // 16):
            off = N - SCCH + v * 16
            if off >= last * C2 + al:
                abuf[pl.ds(off - last * C2, 16)] = tbuf[pl.ds(v * 16, 16)]

        def fstep(j, cy):
            abuf[pl.ds(full_len + j * 16, 16)] = ninf
            return cy

        lax.fori_loop(0, (C2 - full_len) // 16, fstep, jnp.int32(0))

    def kstep(j, cy):
        av = abuf[pl.ds(j * 16, 16)]
        nb = lax.bitcast_convert_type(jnp.float32(0.0) - av, jnp.int32)
        kbuf[pl.ds(j * 16, 16)] = jnp.where(nb < 0, ~nb, nb | jnp.int32(_I32MIN))
        vbuf[pl.ds(j * 16, 16)] = iota + (base + j * 16)
        return cy

    lax.fori_loop(0, C2 // 16, kstep, jnp.int32(0))
    pltpu.sync_copy(kbuf, skA.at[pl.ds(base, C2)])
    pltpu.sync_copy(vbuf, siA.at[pl.ds(base, C2)])

    for p in range(4):
        shift = 8 * p
        srck, srci = (skA, siA) if p % 2 == 0 else (skB, siB)
        dstk, dsti = (skB, siB) if p % 2 == 0 else (skA, siA)
        if p > 0:
            plsc.subcore_barrier()
            pltpu.sync_copy(srck.at[pl.ds(base, C2)], kbuf)
            pltpu.sync_copy(srci.at[pl.ds(base, C2)], vbuf)

        def zstep(j, cy):
            hist[pl.ds(j * 16, 16)] = jnp.zeros((16,), jnp.int32)
            return cy

        lax.fori_loop(0, BINS, zstep, jnp.int32(0))

        def hstep(i, cy):
            pa = iota * SEG2 + 2 * i
            kva = plsc.load_gather(kbuf, [pa])
            kvb = plsc.load_gather(kbuf, [pa + 1])
            plsc.addupdate_scatter(hist, [lanebase + _digit(kva, shift)],
                                   ones)
            plsc.addupdate_scatter(hist, [lanebase + _digit(kvb, shift)],
                                   ones)
            return cy

        lax.fori_loop(0, SEG2 // 2, hstep, jnp.int32(0))

        def rstep(dc, cy):
            acc = jnp.zeros((16,), jnp.int32)
            for l in range(16):
                acc = acc + hist[pl.ds(l * BINS + dc * 16, 16)]
            redb[pl.ds(dc * 16, 16)] = acc
            return cy

        lax.fori_loop(0, BINS // 16, rstep, jnp.int32(0))
        pltpu.sync_copy(redb, cshared.at[pl.ds(w * BINS, BINS)])
        plsc.subcore_barrier()
        pltpu.sync_copy(cshared, credv)

        wvec = jnp.full((16,), w, jnp.int32)
        for dc in range(BINS // 16):
            def wstep(w2, carry):
                tot, wpre = carry
                v = credv[pl.ds(w2 * BINS + dc * 16, 16)]
                keep = jnp.full((16,), w2, jnp.int32) < wvec
                return (tot + v, wpre + jnp.where(keep, v, 0))

            tot, wpre = lax.fori_loop(
                0, NW, wstep,
                (jnp.zeros((16,), jnp.int32), jnp.zeros((16,), jnp.int32)))
            totb[pl.ds(dc * 16, 16)] = tot
            wpreb[pl.ds(dc * 16, 16)] = wpre

        carry = jnp.int32(0)
        for dc in range(BINS // 16):
            v = totb[pl.ds(dc * 16, 16)]
            incl = plsc.cumsum(v)
            baseb[pl.ds(dc * 16, 16)] = (incl - v + wpreb[pl.ds(dc * 16, 16)]
                                         + jnp.full((16,), carry, jnp.int32))
            carry = carry + incl[15]

        def istep(dc, cy):
            run = baseb[pl.ds(dc * 16, 16)]
            for l in range(16):
                sl = pl.ds(l * BINS + dc * 16, 16)
                cnt[sl] = run
                run = run + hist[sl]
            return cy

        lax.fori_loop(0, BINS // 16, istep, jnp.int32(0))

        def sstep(i, cy):
            pa = iota * SEG2 + 2 * i
            pb = pa + 1
            kva = plsc.load_gather(kbuf, [pa])
            kvb = plsc.load_gather(kbuf, [pb])
            cia = lanebase + _digit(kva, shift)
            cib = lanebase + _digit(kvb, shift)
            cura = plsc.load_gather(cnt, [cia])
            curb = plsc.load_gather(cnt, [cib])
            curb = jnp.where(cia == cib, cura + 1, curb)
            plsc.store_scatter(cnt, [cia], cura + 1)
            plsc.store_scatter(cnt, [cib], curb + 1)
            plsc.store_scatter(posbuf, [pa], cura)
            plsc.store_scatter(posbuf, [pb], curb)
            return cy

        lax.fori_loop(0, SEG2 // 2, sstep, jnp.int32(0))

        nch = C2 // SCCH
        kd = [None] * nch
        vd = [None] * nch
        for cc in range(nch):
            slot = slots[cc % NSL]
            if cc >= NSL:
                if kd[cc - NSL] is not None:
                    kd[cc - NSL].wait()
                vd[cc - NSL].wait()
            for v8 in range(SCCH // 16):
                slot[pl.ds(v8 * 16, 16)] = posbuf[pl.ds(cc * SCCH + v8 * 16,
                                                        16)]
            kd[cc] = pltpu.async_copy(kbuf.at[pl.ds(cc * SCCH, SCCH)],
                                      dstk.at[slot], semK)
            vd[cc] = pltpu.async_copy(vbuf.at[pl.ds(cc * SCCH, SCCH)],
                                      dsti.at[slot], semV)
        for cc in range(max(0, nch - NSL), nch):
            if kd[cc] is not None:
                kd[cc].wait()
            vd[cc].wait()

    plsc.subcore_barrier()
    pltpu.sync_copy(siA.at[pl.ds(base, C2)], vbuf)
    pltpu.sync_copy(vbuf, idx_out.at[pl.ds(base, C2)])
    pltpu.sync_copy(skA.at[pl.ds(base, C2)], kbuf)

    neg = jnp.float32(-3.0e38)

    def lstep(j, mv):
        kv = kbuf[pl.ds(j * 16, 16)]
        nb = jnp.where(kv < 0, kv & jnp.int32(0x7FFFFFFF), ~kv)
        av = jnp.float32(0.0) - lax.bitcast_convert_type(nb, jnp.float32)
        abuf[pl.ds(j * 16, 16)] = av
        g = iota + (base + j * 16)
        return jnp.maximum(mv, jnp.where(g < K, av, neg))

    mvec = lax.fori_loop(0, C2 // 16, lstep, jnp.full((16,), neg, jnp.float32))
    m_w = _vmax(mvec)

    def estep(j, sv):
        av = abuf[pl.ds(j * 16, 16)]
        g = iota + (base + j * 16)
        return sv + jnp.where(g < K, jnp.exp(av - m_w), 0.0)

    svec = lax.fori_loop(0, C2 // 16, estep, jnp.zeros((16,), jnp.float32))
    s_w = _vsum(svec)

    redb[pl.ds(0, 16)] = plsc.bitcast(jnp.full((16,), m_w, jnp.float32),
                                      jnp.int32)
    redb[pl.ds(16, 16)] = plsc.bitcast(jnp.full((16,), s_w, jnp.float32),
                                       jnp.int32)
    pltpu.sync_copy(redb, cshared.at[pl.ds(w * BINS, BINS)])
    plsc.subcore_barrier()
    pltpu.sync_copy(cshared, credv)

    gvec = jnp.full((16,), neg, jnp.float32)
    for w2 in range(NW):
        gvec = jnp.maximum(
            gvec, plsc.bitcast(credv[pl.ds(w2 * BINS, 16)], jnp.float32))
    gmax = _vmax(gvec)
    tvec = jnp.zeros((16,), jnp.float32)
    for w2 in range(NW):
        m2 = plsc.bitcast(credv[pl.ds(w2 * BINS, 16)], jnp.float32)
        s2 = plsc.bitcast(credv[pl.ds(w2 * BINS + 16, 16)], jnp.float32)
        tvec = tvec + s2 * jnp.exp(m2 - gmax)
    total = _vmax(tvec)
    inv_vec = jnp.ones((16,), jnp.float32) / jnp.full((16,), total,
                                                      jnp.float32)

    nfull = K // C2
    ktail = K - nfull * C2

    @pl.when(w < nfull)
    def _():
        def qstep(j, cy):
            sl = pl.ds(j * 16, 16)
            abuf[sl] = jnp.exp(abuf[sl] - gmax) * inv_vec
            return cy

        lax.fori_loop(0, C2 // 16, qstep, jnp.int32(0))
        pltpu.sync_copy(abuf, qa_out.at[pl.ds(base, C2)])

    @pl.when(w == nfull)
    def _():
        def qstep(j, cy):
            sl = pl.ds(j * 16, 16)
            qatail[sl] = jnp.exp(abuf[sl] - gmax) * inv_vec
            return cy

        lax.fori_loop(0, ktail // 16, qstep, jnp.int32(0))
        pltpu.sync_copy(qatail, qa_out.at[pl.ds(nfull * C2, ktail)])


def _sort_call():
    scratch = [
        pltpu.VMEM((C2,), jnp.float32),
        pltpu.VMEM((SCCH,), jnp.float32),
        pltpu.VMEM((C2,), jnp.int32),
        pltpu.VMEM((C2,), jnp.int32),
        pltpu.VMEM((BINS * 16,), jnp.int32),
        pltpu.VMEM((BINS * 16,), jnp.int32),
        pltpu.VMEM((BINS,), jnp.int32),
        pltpu.VMEM((BINS,), jnp.int32),
        pltpu.VMEM((BINS,), jnp.int32),
        pltpu.VMEM((BINS,), jnp.int32),
        pltpu.VMEM((NW * BINS,), jnp.int32),
        pltpu.VMEM((C2,), jnp.int32),
        pltpu.VMEM((K - (K // C2) * C2,), jnp.float32),
    ] + [pltpu.VMEM((SCCH,), jnp.int32) for _ in range(NSL)] + [
        pltpu.VMEM_SHARED((NP,), jnp.int32),
        pltpu.VMEM_SHARED((NP,), jnp.int32),
        pltpu.VMEM_SHARED((NP,), jnp.int32),
        pltpu.VMEM_SHARED((NP,), jnp.int32),
        pltpu.VMEM_SHARED((NW * BINS,), jnp.int32),
        pltpu.SemaphoreType.DMA,
        pltpu.SemaphoreType.DMA,
    ]
    return pl.kernel(_sort_body,
                     out_type=(jax.ShapeDtypeStruct((NP,), jnp.int32),
                               jax.ShapeDtypeStruct((K,), jnp.float32)),
                     mesh=_mesh1, compiler_params=_sc_params,
                     scratch_types=scratch)


def _radix_argsort(query_attention):
    return _sort_call()(query_attention)


_gather_call = functools.partial(
    pl.kernel,
    compiler_params=_sc_params,
    out_type=(
        jax.ShapeDtypeStruct((K, D), jnp.float32),
        jax.ShapeDtypeStruct((3 * K,), jnp.float32),
    ),
    mesh=_mesh,
    scratch_types=[
        pltpu.VMEM((R,), jnp.int32),
        pltpu.VMEM((RING, CH, D), jnp.float32),
        pltpu.VMEM((3 * R,), jnp.int32),
        pltpu.VMEM((3 * R,), jnp.float32),
        pltpu.VMEM((3 * R,), jnp.float32),
        pltpu.SemaphoreType.DMA,
        pltpu.SemaphoreType.DMA,
        pltpu.SemaphoreType.DMA,
    ],
)


@jax.jit
def kernel(query_feature, query_points, query_attention):
    idx, qa = _radix_argsort(query_attention)

    qf, qp = _gather_call(_gather_body)(
        idx, query_feature, query_points.reshape(-1))
    return qf, qp.reshape(K, 3), qa


# speedup vs baseline: 1.0559x; 1.0559x over previous
import functools
import jax
import jax.numpy as jnp
from jax import lax
from jax.experimental import pallas as pl
from jax.experimental.pallas import tpu as pltpu
from jax.experimental.pallas import tpu_sc as plsc

N = 100000
K = 50000
D = 128
NC, NS = 2, 16
W = NC * NS
CH = 128
NCH = 13
R = CH * NCH
RING = 5

_mesh = plsc.VectorSubcoreMesh(core_axis_name="c", subcore_axis_name="s")


def _wid():
    return lax.axis_index("s") * NC + lax.axis_index("c")


def _win_start(w):
    return jnp.minimum(w * R, K - R).astype(jnp.int32)


def _vmax(vec):
    r = vec[0]
    for i in range(1, 16):
        r = jnp.maximum(r, vec[i])
    return r


def _vsum(vec):
    r = vec[0]
    for i in range(1, 16):
        r = r + vec[i]
    return r


def _gather_body(idx_hbm, attn_hbm, qf_hbm, qp_hbm, qf_out, qp_out,
                 asel_out, m_out, s_out,
                 idx1d, a1d, rows, qpidx, qpg, qpi, v16,
                 semA, semG, semW, semP):
    w = _wid()
    o = _win_start(w)
    own0 = (w * R).astype(jnp.int32)
    iota = lax.iota(jnp.int32, 16)

    pltpu.sync_copy(idx_hbm.at[pl.ds(o, R)], idx1d)

    ga = [pltpu.async_copy(attn_hbm.at[idx1d.at[pl.ds(c * CH, CH)]],
                           a1d.at[pl.ds(c * CH, CH)], semA)
          for c in range(NCH)]

    def pstep(i, carry):
        v = idx1d[pl.ds(i * 16, 16)] * 3
        qpidx[pl.ds(i * 16, 16)] = v
        qpidx[pl.ds(R + i * 16, 16)] = v + 1
        qpidx[pl.ds(2 * R + i * 16, 16)] = v + 2
        return carry

    lax.fori_loop(0, R // 16, pstep, jnp.int32(0))
    gp = [pltpu.async_copy(
              qp_hbm.at[qpidx.at[pl.ds(comp * R + c * CH, CH)]],
              qpg.at[pl.ds(comp * R + c * CH, CH)], semP)
          for comp in range(3) for c in range(NCH)]

    gq, sq = [None] * NCH, [None] * NCH
    for c in range(4):
        gq[c] = pltpu.async_copy(qf_hbm.at[idx1d.at[pl.ds(c * CH, CH)]],
                                 rows.at[c % RING], semG)
    for c in range(NCH):
        gq[c].wait()
        sq[c] = pltpu.async_copy(rows.at[c % RING],
                                 qf_out.at[pl.ds(o + c * CH, CH)], semW)
        nxt = c + 4
        if nxt < NCH:
            prv = nxt - RING
            if prv >= 0:
                sq[prv].wait()
            gq[nxt] = pltpu.async_copy(
                qf_hbm.at[idx1d.at[pl.ds(nxt * CH, CH)]],
                rows.at[nxt % RING], semG)
    for c in range(max(0, NCH - RING), NCH):
        sq[c].wait()

    for g in gp:
        g.wait()

    def istep(i, carry):
        base = i * 16
        pos = (iota + base) * 3
        plsc.store_scatter(qpi, [pos], qpg[pl.ds(base, 16)])
        plsc.store_scatter(qpi, [pos + 1], qpg[pl.ds(R + base, 16)])
        plsc.store_scatter(qpi, [pos + 2], qpg[pl.ds(2 * R + base, 16)])
        return carry

    lax.fori_loop(0, R // 16, istep, jnp.int32(0))
    pltpu.sync_copy(qpi, qp_out.at[pl.ds(3 * o, 3 * R)])

    for g in ga:
        g.wait()
    neg = jnp.float32(-3.0e38)

    def mstep(i, mv):
        a = a1d[pl.ds(i * 16, 16)]
        g = iota + (o + i * 16)
        return jnp.maximum(mv, jnp.where(g >= own0, a, neg))

    mvec = lax.fori_loop(0, R // 16, mstep, jnp.full((16,), neg, jnp.float32))
    m_w = _vmax(mvec)

    def sstep(i, sv):
        a = a1d[pl.ds(i * 16, 16)]
        g = iota + (o + i * 16)
        return sv + jnp.where(g >= own0, jnp.exp(a - m_w), 0.0)

    svec = lax.fori_loop(0, R // 16, sstep, jnp.zeros((16,), jnp.float32))
    s_w = _vsum(svec)

    v16[...] = jnp.full((16,), m_w, jnp.float32)
    pltpu.sync_copy(v16, m_out.at[pl.ds(w * 16, 16)])
    v16[...] = jnp.full((16,), s_w, jnp.float32)
    pltpu.sync_copy(v16, s_out.at[pl.ds(w * 16, 16)])
    pltpu.sync_copy(a1d, asel_out.at[pl.ds(w * R, R)])


def _norm_body(asel_hbm, m_hbm, s_hbm, qa_out, a1d, mb, sb, sem):
    w = _wid()
    o = _win_start(w)
    pltpu.sync_copy(m_hbm, mb)
    pltpu.sync_copy(s_hbm, sb)
    pltpu.sync_copy(asel_hbm.at[pl.ds(w * R, R)], a1d)

    neg = jnp.float32(-3.0e38)
    gvec = jnp.full((16,), neg, jnp.float32)
    for i in range(W):
        gvec = jnp.maximum(gvec, mb[pl.ds(i * 16, 16)])
    gmax = _vmax(gvec)

    tvec = jnp.zeros((16,), jnp.float32)
    for i in range(W):
        tvec = tvec + sb[pl.ds(i * 16, 16)] * jnp.exp(mb[pl.ds(i * 16, 16)] - gmax)
    total = _vmax(tvec)
    inv_vec = jnp.ones((16,), jnp.float32) / jnp.full((16,), total, jnp.float32)

    def qstep(i, carry):
        sl = pl.ds(i * 16, 16)
        a1d[sl] = jnp.exp(a1d[sl] - gmax) * inv_vec
        return carry

    lax.fori_loop(0, R // 16, qstep, jnp.int32(0))
    pltpu.sync_copy(a1d, qa_out.at[pl.ds(o, R)])


_sc_params = pltpu.CompilerParams(needs_layout_passes=False)

NP = 102400
NW = 16
C2 = NP // NW
SEG2 = C2 // 16
BINS = 256
SCCH = 128
NSL = 8

_I32MIN = -2147483648

_mesh1 = plsc.VectorSubcoreMesh(core_axis_name="c", subcore_axis_name="s",
                                num_cores=1)


def _digit(kv, shift):
    d = kv
    if shift:
        d = plsc.bitcast(
            plsc.bitcast(d, jnp.uint32) >> jnp.uint32(shift), jnp.int32)
    return d & jnp.int32(BINS - 1)


def _sort_body(attn_hbm, idx_out,
               abuf, tbuf, kbuf, vbuf, hist, cnt, redb, totb, wpreb, baseb,
               credv, posbuf, s0, s1, s2, s3, s4, s5, s6, s7,
               skA, siA, skB, siB, cshared, semK, semV):
    slots = [s0, s1, s2, s3, s4, s5, s6, s7]
    w = lax.axis_index("s")
    base = w * C2
    iota = lax.iota(jnp.int32, 16)
    ones = jnp.ones((16,), jnp.int32)
    lanebase = iota * BINS

    last = NW - 1
    full_len = N - last * C2
    al = (full_len // SCCH) * SCCH

    @pl.when(w < last)
    def _():
        pltpu.sync_copy(attn_hbm.at[pl.ds(base, C2)], abuf)

    @pl.when(w == last)
    def _():
        pltpu.sync_copy(attn_hbm.at[pl.ds(last * C2, al)],
                        abuf.at[pl.ds(0, al)])
        pltpu.sync_copy(attn_hbm.at[pl.ds(N - SCCH, SCCH)], tbuf)
        ninf = jnp.full((16,), -jnp.inf, jnp.float32)
        for v in range(SCCH // 16):
            off = N - SCCH + v * 16
            if off >= last * C2 + al:
                abuf[pl.ds(off - last * C2, 16)] = tbuf[pl.ds(v * 16, 16)]

        def fstep(j, cy):
            abuf[pl.ds(full_len + j * 16, 16)] = ninf
            return cy

        lax.fori_loop(0, (C2 - full_len) // 16, fstep, jnp.int32(0))

    def kstep(j, cy):
        av = abuf[pl.ds(j * 16, 16)]
        nb = lax.bitcast_convert_type(jnp.float32(0.0) - av, jnp.int32)
        kbuf[pl.ds(j * 16, 16)] = jnp.where(nb < 0, ~nb, nb | jnp.int32(_I32MIN))
        vbuf[pl.ds(j * 16, 16)] = iota + (base + j * 16)
        return cy

    lax.fori_loop(0, C2 // 16, kstep, jnp.int32(0))
    pltpu.sync_copy(kbuf, skA.at[pl.ds(base, C2)])
    pltpu.sync_copy(vbuf, siA.at[pl.ds(base, C2)])

    for p in range(4):
        shift = 8 * p
        srck, srci = (skA, siA) if p % 2 == 0 else (skB, siB)
        dstk, dsti = (skB, siB) if p % 2 == 0 else (skA, siA)
        if p > 0:
            plsc.subcore_barrier()
            pltpu.sync_copy(srck.at[pl.ds(base, C2)], kbuf)
            pltpu.sync_copy(srci.at[pl.ds(base, C2)], vbuf)

        def zstep(j, cy):
            hist[pl.ds(j * 16, 16)] = jnp.zeros((16,), jnp.int32)
            return cy

        lax.fori_loop(0, BINS, zstep, jnp.int32(0))

        def hstep(i, cy):
            pa = iota * SEG2 + 2 * i
            kva = plsc.load_gather(kbuf, [pa])
            kvb = plsc.load_gather(kbuf, [pa + 1])
            plsc.addupdate_scatter(hist, [lanebase + _digit(kva, shift)],
                                   ones)
            plsc.addupdate_scatter(hist, [lanebase + _digit(kvb, shift)],
                                   ones)
            return cy

        lax.fori_loop(0, SEG2 // 2, hstep, jnp.int32(0))

        def rstep(dc, cy):
            acc = jnp.zeros((16,), jnp.int32)
            for l in range(16):
                acc = acc + hist[pl.ds(l * BINS + dc * 16, 16)]
            redb[pl.ds(dc * 16, 16)] = acc
            return cy

        lax.fori_loop(0, BINS // 16, rstep, jnp.int32(0))
        pltpu.sync_copy(redb, cshared.at[pl.ds(w * BINS, BINS)])
        plsc.subcore_barrier()
        pltpu.sync_copy(cshared, credv)

        wvec = jnp.full((16,), w, jnp.int32)
        for dc in range(BINS // 16):
            def wstep(w2, carry):
                tot, wpre = carry
                v = credv[pl.ds(w2 * BINS + dc * 16, 16)]
                keep = jnp.full((16,), w2, jnp.int32) < wvec
                return (tot + v, wpre + jnp.where(keep, v, 0))

            tot, wpre = lax.fori_loop(
                0, NW, wstep,
                (jnp.zeros((16,), jnp.int32), jnp.zeros((16,), jnp.int32)))
            totb[pl.ds(dc * 16, 16)] = tot
            wpreb[pl.ds(dc * 16, 16)] = wpre

        carry = jnp.int32(0)
        for dc in range(BINS // 16):
            v = totb[pl.ds(dc * 16, 16)]
            incl = plsc.cumsum(v)
            baseb[pl.ds(dc * 16, 16)] = (incl - v + wpreb[pl.ds(dc * 16, 16)]
                                         + jnp.full((16,), carry, jnp.int32))
            carry = carry + incl[15]

        def istep(dc, cy):
            run = baseb[pl.ds(dc * 16, 16)]
            for l in range(16):
                sl = pl.ds(l * BINS + dc * 16, 16)
                cnt[sl] = run
                run = run + hist[sl]
            return cy

        lax.fori_loop(0, BINS // 16, istep, jnp.int32(0))

        def sstep(i, cy):
            pa = iota * SEG2 + 2 * i
            pb = pa + 1
            kva = plsc.load_gather(kbuf, [pa])
            kvb = plsc.load_gather(kbuf, [pb])
            cia = lanebase + _digit(kva, shift)
            cib = lanebase + _digit(kvb, shift)
            cura = plsc.load_gather(cnt, [cia])
            curb = plsc.load_gather(cnt, [cib])
            curb = jnp.where(cia == cib, cura + 1, curb)
            plsc.store_scatter(cnt, [cia], cura + 1)
            plsc.store_scatter(cnt, [cib], curb + 1)
            plsc.store_scatter(posbuf, [pa], cura)
            plsc.store_scatter(posbuf, [pb], curb)
            return cy

        lax.fori_loop(0, SEG2 // 2, sstep, jnp.int32(0))

        nch = C2 // SCCH
        kd = [None] * nch
        vd = [None] * nch
        for cc in range(nch):
            slot = slots[cc % NSL]
            if cc >= NSL:
                if kd[cc - NSL] is not None:
                    kd[cc - NSL].wait()
                vd[cc - NSL].wait()
            for v8 in range(SCCH // 16):
                slot[pl.ds(v8 * 16, 16)] = posbuf[pl.ds(cc * SCCH + v8 * 16,
                                                        16)]
            if p < 3:
                kd[cc] = pltpu.async_copy(kbuf.at[pl.ds(cc * SCCH, SCCH)],
                                          dstk.at[slot], semK)
            vd[cc] = pltpu.async_copy(vbuf.at[pl.ds(cc * SCCH, SCCH)],
                                      dsti.at[slot], semV)
        for cc in range(max(0, nch - NSL), nch):
            if kd[cc] is not None:
                kd[cc].wait()
            vd[cc].wait()

    plsc.subcore_barrier()
    pltpu.sync_copy(siA.at[pl.ds(base, C2)], vbuf)
    pltpu.sync_copy(vbuf, idx_out.at[pl.ds(base, C2)])


def _sort_call():
    scratch = [
        pltpu.VMEM((C2,), jnp.float32),
        pltpu.VMEM((SCCH,), jnp.float32),
        pltpu.VMEM((C2,), jnp.int32),
        pltpu.VMEM((C2,), jnp.int32),
        pltpu.VMEM((BINS * 16,), jnp.int32),
        pltpu.VMEM((BINS * 16,), jnp.int32),
        pltpu.VMEM((BINS,), jnp.int32),
        pltpu.VMEM((BINS,), jnp.int32),
        pltpu.VMEM((BINS,), jnp.int32),
        pltpu.VMEM((BINS,), jnp.int32),
        pltpu.VMEM((NW * BINS,), jnp.int32),
        pltpu.VMEM((C2,), jnp.int32),
    ] + [pltpu.VMEM((SCCH,), jnp.int32) for _ in range(NSL)] + [
        pltpu.VMEM_SHARED((NP,), jnp.int32),
        pltpu.VMEM_SHARED((NP,), jnp.int32),
        pltpu.VMEM_SHARED((NP,), jnp.int32),
        pltpu.VMEM_SHARED((NP,), jnp.int32),
        pltpu.VMEM_SHARED((NW * BINS,), jnp.int32),
        pltpu.SemaphoreType.DMA,
        pltpu.SemaphoreType.DMA,
    ]
    return pl.kernel(_sort_body,
                     out_type=jax.ShapeDtypeStruct((NP,), jnp.int32),
                     mesh=_mesh1, compiler_params=_sc_params,
                     scratch_types=scratch)


def _radix_argsort(query_attention):
    return _sort_call()(query_attention)


_gather_call = functools.partial(
    pl.kernel,
    compiler_params=_sc_params,
    out_type=(
        jax.ShapeDtypeStruct((K, D), jnp.float32),
        jax.ShapeDtypeStruct((3 * K,), jnp.float32),
        jax.ShapeDtypeStruct((W * R,), jnp.float32),
        jax.ShapeDtypeStruct((W * 16,), jnp.float32),
        jax.ShapeDtypeStruct((W * 16,), jnp.float32),
    ),
    mesh=_mesh,
    scratch_types=[
        pltpu.VMEM((R,), jnp.int32),
        pltpu.VMEM((R,), jnp.float32),
        pltpu.VMEM((RING, CH, D), jnp.float32),
        pltpu.VMEM((3 * R,), jnp.int32),
        pltpu.VMEM((3 * R,), jnp.float32),
        pltpu.VMEM((3 * R,), jnp.float32),
        pltpu.VMEM((16,), jnp.float32),
        pltpu.SemaphoreType.DMA,
        pltpu.SemaphoreType.DMA,
        pltpu.SemaphoreType.DMA,
        pltpu.SemaphoreType.DMA,
    ],
)

_norm_call = functools.partial(
    pl.kernel,
    compiler_params=_sc_params,
    out_type=jax.ShapeDtypeStruct((K,), jnp.float32),
    mesh=_mesh,
    scratch_types=[
        pltpu.VMEM((R,), jnp.float32),
        pltpu.VMEM((W * 16,), jnp.float32),
        pltpu.VMEM((W * 16,), jnp.float32),
        pltpu.SemaphoreType.DMA,
    ],
)


@jax.jit
def kernel(query_feature, query_points, query_attention):
    idx = _radix_argsort(query_attention)

    qf, qp, asel, m, s = _gather_call(_gather_body)(
        idx, query_attention, query_feature, query_points.reshape(-1))
    qa = _norm_call(_norm_body)(asel, m, s)
    return qf, qp.reshape(K, 3), qa


# speedup vs baseline: 1.0603x; 1.0042x over previous
import functools
import jax
import jax.numpy as jnp
from jax import lax
from jax.experimental import pallas as pl
from jax.experimental.pallas import tpu as pltpu
from jax.experimental.pallas import tpu_sc as plsc

N = 100000
K = 50000
D = 128
NC, NS = 2, 16
W = NC * NS
CH = 128
NCH = 13
R = CH * NCH
RING = 6

_mesh = plsc.VectorSubcoreMesh(core_axis_name="c", subcore_axis_name="s")


def _wid():
    return lax.axis_index("s") * NC + lax.axis_index("c")


def _win_start(w):
    return jnp.minimum(w * R, K - R).astype(jnp.int32)


def _vmax(vec):
    r = vec[0]
    for i in range(1, 16):
        r = jnp.maximum(r, vec[i])
    return r


def _vsum(vec):
    r = vec[0]
    for i in range(1, 16):
        r = r + vec[i]
    return r


def _gather_body(idx_hbm, attn_hbm, qf_hbm, qp_hbm, qf_out, qp_out,
                 asel_out, m_out, s_out,
                 idx1d, a1d, rows, qpidx, qpg, qpi, v16,
                 semA, semG, semW, semP):
    w = _wid()
    o = _win_start(w)
    own0 = (w * R).astype(jnp.int32)
    iota = lax.iota(jnp.int32, 16)

    pltpu.sync_copy(idx_hbm.at[pl.ds(o, R)], idx1d)

    ga = [pltpu.async_copy(attn_hbm.at[idx1d.at[pl.ds(c * CH, CH)]],
                           a1d.at[pl.ds(c * CH, CH)], semA)
          for c in range(NCH)]

    def pstep(i, carry):
        v = idx1d[pl.ds(i * 16, 16)] * 3
        qpidx[pl.ds(i * 16, 16)] = v
        qpidx[pl.ds(R + i * 16, 16)] = v + 1
        qpidx[pl.ds(2 * R + i * 16, 16)] = v + 2
        return carry

    lax.fori_loop(0, R // 16, pstep, jnp.int32(0))
    gp = [pltpu.async_copy(
              qp_hbm.at[qpidx.at[pl.ds(comp * R + c * CH, CH)]],
              qpg.at[pl.ds(comp * R + c * CH, CH)], semP)
          for comp in range(3) for c in range(NCH)]

    gq, sq = [None] * NCH, [None] * NCH
    for c in range(4):
        gq[c] = pltpu.async_copy(qf_hbm.at[idx1d.at[pl.ds(c * CH, CH)]],
                                 rows.at[c % RING], semG)
    for c in range(NCH):
        gq[c].wait()
        sq[c] = pltpu.async_copy(rows.at[c % RING],
                                 qf_out.at[pl.ds(o + c * CH, CH)], semW)
        nxt = c + 4
        if nxt < NCH:
            prv = nxt - RING
            if prv >= 0:
                sq[prv].wait()
            gq[nxt] = pltpu.async_copy(
                qf_hbm.at[idx1d.at[pl.ds(nxt * CH, CH)]],
                rows.at[nxt % RING], semG)
    for c in range(max(0, NCH - RING), NCH):
        sq[c].wait()

    for g in gp:
        g.wait()

    def istep(i, carry):
        base = i * 16
        pos = (iota + base) * 3
        plsc.store_scatter(qpi, [pos], qpg[pl.ds(base, 16)])
        plsc.store_scatter(qpi, [pos + 1], qpg[pl.ds(R + base, 16)])
        plsc.store_scatter(qpi, [pos + 2], qpg[pl.ds(2 * R + base, 16)])
        return carry

    lax.fori_loop(0, R // 16, istep, jnp.int32(0))
    pltpu.sync_copy(qpi, qp_out.at[pl.ds(3 * o, 3 * R)])

    for g in ga:
        g.wait()
    neg = jnp.float32(-3.0e38)

    def mstep(i, mv):
        a = a1d[pl.ds(i * 16, 16)]
        g = iota + (o + i * 16)
        return jnp.maximum(mv, jnp.where(g >= own0, a, neg))

    mvec = lax.fori_loop(0, R // 16, mstep, jnp.full((16,), neg, jnp.float32))
    m_w = _vmax(mvec)

    def sstep(i, sv):
        a = a1d[pl.ds(i * 16, 16)]
        g = iota + (o + i * 16)
        return sv + jnp.where(g >= own0, jnp.exp(a - m_w), 0.0)

    svec = lax.fori_loop(0, R // 16, sstep, jnp.zeros((16,), jnp.float32))
    s_w = _vsum(svec)

    v16[...] = jnp.full((16,), m_w, jnp.float32)
    pltpu.sync_copy(v16, m_out.at[pl.ds(w * 16, 16)])
    v16[...] = jnp.full((16,), s_w, jnp.float32)
    pltpu.sync_copy(v16, s_out.at[pl.ds(w * 16, 16)])
    pltpu.sync_copy(a1d, asel_out.at[pl.ds(w * R, R)])


def _norm_body(asel_hbm, m_hbm, s_hbm, qa_out, a1d, mb, sb, sem):
    w = _wid()
    o = _win_start(w)
    pltpu.sync_copy(m_hbm, mb)
    pltpu.sync_copy(s_hbm, sb)
    pltpu.sync_copy(asel_hbm.at[pl.ds(w * R, R)], a1d)

    neg = jnp.float32(-3.0e38)
    gvec = jnp.full((16,), neg, jnp.float32)
    for i in range(W):
        gvec = jnp.maximum(gvec, mb[pl.ds(i * 16, 16)])
    gmax = _vmax(gvec)

    tvec = jnp.zeros((16,), jnp.float32)
    for i in range(W):
        tvec = tvec + sb[pl.ds(i * 16, 16)] * jnp.exp(mb[pl.ds(i * 16, 16)] - gmax)
    total = _vmax(tvec)
    inv_vec = jnp.ones((16,), jnp.float32) / jnp.full((16,), total, jnp.float32)

    def qstep(i, carry):
        sl = pl.ds(i * 16, 16)
        a1d[sl] = jnp.exp(a1d[sl] - gmax) * inv_vec
        return carry

    lax.fori_loop(0, R // 16, qstep, jnp.int32(0))
    pltpu.sync_copy(a1d, qa_out.at[pl.ds(o, R)])


_sc_params = pltpu.CompilerParams(needs_layout_passes=False)

NP = 102400
NW = 16
C2 = NP // NW
SEG2 = C2 // 16
BINS = 256
SCCH = 128
NSL = 8

_I32MIN = -2147483648

_mesh1 = plsc.VectorSubcoreMesh(core_axis_name="c", subcore_axis_name="s",
                                num_cores=1)


def _digit(kv, shift):
    d = kv
    if shift:
        d = plsc.bitcast(
            plsc.bitcast(d, jnp.uint32) >> jnp.uint32(shift), jnp.int32)
    return d & jnp.int32(BINS - 1)


def _sort_body(attn_hbm, idx_out,
               abuf, tbuf, kbuf, vbuf, hist, cnt, redb, totb, wpreb, baseb,
               credv, posbuf, s0, s1, s2, s3, s4, s5, s6, s7,
               skA, siA, skB, siB, cshared, semK, semV):
    slots = [s0, s1, s2, s3, s4, s5, s6, s7]
    w = lax.axis_index("s")
    base = w * C2
    iota = lax.iota(jnp.int32, 16)
    ones = jnp.ones((16,), jnp.int32)
    lanebase = iota * BINS

    last = NW - 1
    full_len = N - last * C2
    al = (full_len // SCCH) * SCCH

    @pl.when(w < last)
    def _():
        pltpu.sync_copy(attn_hbm.at[pl.ds(base, C2)], abuf)

    @pl.when(w == last)
    def _():
        pltpu.sync_copy(attn_hbm.at[pl.ds(last * C2, al)],
                        abuf.at[pl.ds(0, al)])
        pltpu.sync_copy(attn_hbm.at[pl.ds(N - SCCH, SCCH)], tbuf)
        ninf = jnp.full((16,), -jnp.inf, jnp.float32)
        for v in range(SCCH // 16):
            off = N - SCCH + v * 16
            if off >= last * C2 + al:
                abuf[pl.ds(off - last * C2, 16)] = tbuf[pl.ds(v * 16, 16)]

        def fstep(j, cy):
            abuf[pl.ds(full_len + j * 16, 16)] = ninf
            return cy

        lax.fori_loop(0, (C2 - full_len) // 16, fstep, jnp.int32(0))

    def kstep(j, cy):
        av = abuf[pl.ds(j * 16, 16)]
        nb = lax.bitcast_convert_type(jnp.float32(0.0) - av, jnp.int32)
        kbuf[pl.ds(j * 16, 16)] = jnp.where(nb < 0, ~nb, nb | jnp.int32(_I32MIN))
        vbuf[pl.ds(j * 16, 16)] = iota + (base + j * 16)
        return cy

    lax.fori_loop(0, C2 // 16, kstep, jnp.int32(0))
    pltpu.sync_copy(kbuf, skA.at[pl.ds(base, C2)])
    pltpu.sync_copy(vbuf, siA.at[pl.ds(base, C2)])

    for p in range(4):
        shift = 8 * p
        srck, srci = (skA, siA) if p % 2 == 0 else (skB, siB)
        dstk, dsti = (skB, siB) if p % 2 == 0 else (skA, siA)
        if p > 0:
            plsc.subcore_barrier()
            pltpu.sync_copy(srck.at[pl.ds(base, C2)], kbuf)
            pltpu.sync_copy(srci.at[pl.ds(base, C2)], vbuf)

        def zstep(j, cy):
            hist[pl.ds(j * 16, 16)] = jnp.zeros((16,), jnp.int32)
            return cy

        lax.fori_loop(0, BINS, zstep, jnp.int32(0))

        def hstep(i, cy):
            pa = iota * SEG2 + 4 * i
            for u in range(4):
                kv = plsc.load_gather(kbuf, [pa + u])
                plsc.addupdate_scatter(hist, [lanebase + _digit(kv, shift)],
                                       ones)
            return cy

        lax.fori_loop(0, SEG2 // 4, hstep, jnp.int32(0))

        def rstep(dc, cy):
            acc = jnp.zeros((16,), jnp.int32)
            for l in range(16):
                acc = acc + hist[pl.ds(l * BINS + dc * 16, 16)]
            redb[pl.ds(dc * 16, 16)] = acc
            return cy

        lax.fori_loop(0, BINS // 16, rstep, jnp.int32(0))
        pltpu.sync_copy(redb, cshared.at[pl.ds(w * BINS, BINS)])
        plsc.subcore_barrier()
        pltpu.sync_copy(cshared, credv)

        wvec = jnp.full((16,), w, jnp.int32)
        for dc in range(BINS // 16):
            def wstep(w2, carry):
                tot, wpre = carry
                v = credv[pl.ds(w2 * BINS + dc * 16, 16)]
                keep = jnp.full((16,), w2, jnp.int32) < wvec
                return (tot + v, wpre + jnp.where(keep, v, 0))

            tot, wpre = lax.fori_loop(
                0, NW, wstep,
                (jnp.zeros((16,), jnp.int32), jnp.zeros((16,), jnp.int32)))
            totb[pl.ds(dc * 16, 16)] = tot
            wpreb[pl.ds(dc * 16, 16)] = wpre

        carry = jnp.int32(0)
        for dc in range(BINS // 16):
            v = totb[pl.ds(dc * 16, 16)]
            incl = plsc.cumsum(v)
            baseb[pl.ds(dc * 16, 16)] = (incl - v + wpreb[pl.ds(dc * 16, 16)]
                                         + jnp.full((16,), carry, jnp.int32))
            carry = carry + incl[15]

        def istep(dc, cy):
            run = baseb[pl.ds(dc * 16, 16)]
            for l in range(16):
                sl = pl.ds(l * BINS + dc * 16, 16)
                cnt[sl] = run
                run = run + hist[sl]
            return cy

        lax.fori_loop(0, BINS // 16, istep, jnp.int32(0))

        def sstep(i, cy):
            p0 = iota * SEG2 + 4 * i
            ps = [p0, p0 + 1, p0 + 2, p0 + 3]
            cis = [lanebase + _digit(plsc.load_gather(kbuf, [p]), shift)
                   for p in ps]
            curs = [plsc.load_gather(cnt, [ci]) for ci in cis]
            one = jnp.int32(1)
            for j in range(4):
                add = jnp.zeros((16,), jnp.int32)
                for j2 in range(j):
                    add = add + jnp.where(cis[j] == cis[j2], one, 0)
                curs[j] = curs[j] + add
            for j in range(4):
                plsc.store_scatter(cnt, [cis[j]], curs[j] + 1)
                plsc.store_scatter(posbuf, [ps[j]], curs[j])
            return cy

        lax.fori_loop(0, SEG2 // 4, sstep, jnp.int32(0))

        nch = C2 // SCCH
        kd = [None] * nch
        vd = [None] * nch
        for cc in range(nch):
            slot = slots[cc % NSL]
            if cc >= NSL:
                if kd[cc - NSL] is not None:
                    kd[cc - NSL].wait()
                vd[cc - NSL].wait()
            for v8 in range(SCCH // 16):
                slot[pl.ds(v8 * 16, 16)] = posbuf[pl.ds(cc * SCCH + v8 * 16,
                                                        16)]
            if p < 3:
                kd[cc] = pltpu.async_copy(kbuf.at[pl.ds(cc * SCCH, SCCH)],
                                          dstk.at[slot], semK)
            vd[cc] = pltpu.async_copy(vbuf.at[pl.ds(cc * SCCH, SCCH)],
                                      dsti.at[slot], semV)
        for cc in range(max(0, nch - NSL), nch):
            if kd[cc] is not None:
                kd[cc].wait()
            vd[cc].wait()

    plsc.subcore_barrier()
    pltpu.sync_copy(siA.at[pl.ds(base, C2)], vbuf)
    pltpu.sync_copy(vbuf, idx_out.at[pl.ds(base, C2)])


def _sort_call():
    scratch = [
        pltpu.VMEM((C2,), jnp.float32),
        pltpu.VMEM((SCCH,), jnp.float32),
        pltpu.VMEM((C2,), jnp.int32),
        pltpu.VMEM((C2,), jnp.int32),
        pltpu.VMEM((BINS * 16,), jnp.int32),
        pltpu.VMEM((BINS * 16,), jnp.int32),
        pltpu.VMEM((BINS,), jnp.int32),
        pltpu.VMEM((BINS,), jnp.int32),
        pltpu.VMEM((BINS,), jnp.int32),
        pltpu.VMEM((BINS,), jnp.int32),
        pltpu.VMEM((NW * BINS,), jnp.int32),
        pltpu.VMEM((C2,), jnp.int32),
    ] + [pltpu.VMEM((SCCH,), jnp.int32) for _ in range(NSL)] + [
        pltpu.VMEM_SHARED((NP,), jnp.int32),
        pltpu.VMEM_SHARED((NP,), jnp.int32),
        pltpu.VMEM_SHARED((NP,), jnp.int32),
        pltpu.VMEM_SHARED((NP,), jnp.int32),
        pltpu.VMEM_SHARED((NW * BINS,), jnp.int32),
        pltpu.SemaphoreType.DMA,
        pltpu.SemaphoreType.DMA,
    ]
    return pl.kernel(_sort_body,
                     out_type=jax.ShapeDtypeStruct((NP,), jnp.int32),
                     mesh=_mesh1, compiler_params=_sc_params,
                     scratch_types=scratch)


def _radix_argsort(query_attention):
    return _sort_call()(query_attention)


_gather_call = functools.partial(
    pl.kernel,
    compiler_params=_sc_params,
    out_type=(
        jax.ShapeDtypeStruct((K, D), jnp.float32),
        jax.ShapeDtypeStruct((3 * K,), jnp.float32),
        jax.ShapeDtypeStruct((W * R,), jnp.float32),
        jax.ShapeDtypeStruct((W * 16,), jnp.float32),
        jax.ShapeDtypeStruct((W * 16,), jnp.float32),
    ),
    mesh=_mesh,
    scratch_types=[
        pltpu.VMEM((R,), jnp.int32),
        pltpu.VMEM((R,), jnp.float32),
        pltpu.VMEM((RING, CH, D), jnp.float32),
        pltpu.VMEM((3 * R,), jnp.int32),
        pltpu.VMEM((3 * R,), jnp.float32),
        pltpu.VMEM((3 * R,), jnp.float32),
        pltpu.VMEM((16,), jnp.float32),
        pltpu.SemaphoreType.DMA,
        pltpu.SemaphoreType.DMA,
        pltpu.SemaphoreType.DMA,
        pltpu.SemaphoreType.DMA,
    ],
)

_norm_call = functools.partial(
    pl.kernel,
    compiler_params=_sc_params,
    out_type=jax.ShapeDtypeStruct((K,), jnp.float32),
    mesh=_mesh,
    scratch_types=[
        pltpu.VMEM((R,), jnp.float32),
        pltpu.VMEM((W * 16,), jnp.float32),
        pltpu.VMEM((W * 16,), jnp.float32),
        pltpu.SemaphoreType.DMA,
    ],
)


@jax.jit
def kernel(query_feature, query_points, query_attention):
    idx = _radix_argsort(query_attention)

    qf, qp, asel, m, s = _gather_call(_gather_body)(
        idx, query_attention, query_feature, query_points.reshape(-1))
    qa = _norm_call(_norm_body)(asel, m, s)
    return qf, qp.reshape(K, 3), qa
